# baseline (speedup 1.0000x reference)
import jax
import jax.numpy as jnp
from jax.experimental import pallas as pl
from jax.experimental.pallas import tpu as pltpu

_NUM_STRIPES = 4


def _dma_copy(in_hbm, out_hbm, sems):
    rows = in_hbm.shape[0]
    stripe = rows // _NUM_STRIPES
    copies = [
        pltpu.make_async_copy(
            in_hbm.at[pl.ds(k * stripe, stripe), :],
            out_hbm.at[pl.ds(k * stripe, stripe), :],
            sems.at[k],
        )
        for k in range(_NUM_STRIPES)
    ]
    for c in copies:
        c.start()
    for c in copies:
        c.wait()


def kernel(features):
    return pl.pallas_call(
        _dma_copy,
        in_specs=[pl.BlockSpec(memory_space=pltpu.MemorySpace.HBM)],
        out_specs=pl.BlockSpec(memory_space=pltpu.MemorySpace.HBM),
        out_shape=jax.ShapeDtypeStruct(features.shape, features.dtype),
        scratch_shapes=[pltpu.SemaphoreType.DMA((_NUM_STRIPES,))],
    )(features)


# speedup vs baseline: 29.5214x; 29.5214x over previous
import jax
import jax.numpy as jnp
from jax.experimental import pallas as pl

_BLOCK_ROWS = 8


def _copy_block(in_ref, out_ref):
    out_ref[...] = in_ref[...]


def kernel(features):
    rows, cols = features.shape
    return pl.pallas_call(
        _copy_block,
        grid=(rows // _BLOCK_ROWS,),
        in_specs=[pl.BlockSpec((_BLOCK_ROWS, cols), lambda i: (i, 0))],
        out_specs=pl.BlockSpec((_BLOCK_ROWS, cols), lambda i: (i, 0)),
        out_shape=jax.ShapeDtypeStruct((rows, cols), features.dtype),
    )(features)


# speedup vs baseline: 41.8740x; 1.4184x over previous
import jax
import jax.numpy as jnp
from jax.experimental import pallas as pl

_BLOCK_ROWS = 32


def _copy_block(in_ref, out_ref):
    out_ref[...] = in_ref[...]


def kernel(features):
    rows, cols = features.shape
    return pl.pallas_call(
        _copy_block,
        grid=(rows // _BLOCK_ROWS,),
        in_specs=[pl.BlockSpec((_BLOCK_ROWS, cols), lambda i: (i, 0))],
        out_specs=pl.BlockSpec((_BLOCK_ROWS, cols), lambda i: (i, 0)),
        out_shape=jax.ShapeDtypeStruct((rows, cols), features.dtype),
    )(features)


# speedup vs baseline: 47.0450x; 1.1235x over previous
import jax
import jax.numpy as jnp
from jax.experimental import pallas as pl

_BLOCK_ROWS = 64


def _copy_block(in_ref, out_ref):
    out_ref[...] = in_ref[...]


def kernel(features):
    rows, cols = features.shape
    return pl.pallas_call(
        _copy_block,
        grid=(rows // _BLOCK_ROWS,),
        in_specs=[pl.BlockSpec((_BLOCK_ROWS, cols), lambda i: (i, 0))],
        out_specs=pl.BlockSpec((_BLOCK_ROWS, cols), lambda i: (i, 0)),
        out_shape=jax.ShapeDtypeStruct((rows, cols), features.dtype),
    )(features)
